# baseline (speedup 1.0000x reference)
import functools

import jax
import jax.numpy as jnp
from jax import lax
from jax.experimental import pallas as pl
from jax.experimental.pallas import tpu as pltpu
from jax.experimental.pallas import tpu_sc as plsc

VOCAB = 100000
D_MODEL = 1024
NUM_CORES = 2
NUM_SUBCORES = 16
NUM_WORKERS = NUM_CORES * NUM_SUBCORES

CHUNK = 8
NBUF = 8


def _embed_body(n_rows, x_hbm, table_hbm, out_hbm, idx_v, rows_v, gsems, psems):
    b_per_w = n_rows // NUM_WORKERS
    n_chunks = b_per_w // CHUNK
    wid = lax.axis_index("s") * NUM_CORES + lax.axis_index("c")
    base = wid * b_per_w
    pltpu.sync_copy(x_hbm.at[pl.ds(base, b_per_w)], idx_v)

    def gather(ch, b):
        pltpu.async_copy(
            table_hbm.at[idx_v.at[pl.ds(ch * CHUNK, CHUNK)]],
            rows_v.at[b], gsems.at[b],
        )

    def put(ch, b):
        pltpu.async_copy(
            rows_v.at[b], out_hbm.at[pl.ds(base + ch * CHUNK, CHUNK)],
            psems.at[b],
        )

    for b in range(NBUF):
        gather(b, b)

    @pl.loop(0, n_chunks, step=NBUF)
    def _chunks(c0):
        def wait_gather_issue_put(b):
            ch = c0 + b
            pltpu.make_async_copy(
                table_hbm.at[idx_v.at[pl.ds(ch * CHUNK, CHUNK)]],
                rows_v.at[b], gsems.at[b],
            ).wait()
            put(ch, b)

        def wait_put_issue_gather(b):
            ch = c0 + b
            pltpu.make_async_copy(
                rows_v.at[b], out_hbm.at[pl.ds(base + ch * CHUNK, CHUNK)],
                psems.at[b],
            ).wait()

            @pl.when(ch + NBUF < n_chunks)
            def _():
                gather(ch + NBUF, b)

        OFFSET = 4
        for step in range(NBUF + OFFSET):
            if step < NBUF:
                wait_gather_issue_put(step)
            if step >= OFFSET:
                wait_put_issue_gather(step - OFFSET)


def kernel(x, table):
    B, T = x.shape
    n_rows = B * T
    x_flat = x.reshape(n_rows).astype(jnp.int32)

    mesh = plsc.VectorSubcoreMesh(
        core_axis_name="c", subcore_axis_name="s",
        num_cores=NUM_CORES, num_subcores=NUM_SUBCORES,
    )
    b_per_w = n_rows // NUM_WORKERS
    run = pl.kernel(
        functools.partial(_embed_body, n_rows),
        out_type=jax.ShapeDtypeStruct((n_rows, D_MODEL), jnp.float32),
        mesh=mesh,
        scratch_types=[
            pltpu.VMEM((b_per_w,), jnp.int32),
            pltpu.VMEM((NBUF, CHUNK, D_MODEL), jnp.float32),
            pltpu.SemaphoreType.DMA((NBUF,)),
            pltpu.SemaphoreType.DMA((NBUF,)),
        ],
    )
    out = run(x_flat, table)
    return out.reshape(B, T, D_MODEL)


# speedup vs baseline: 1.0095x; 1.0095x over previous
import functools

import jax
import jax.numpy as jnp
from jax import lax
from jax.experimental import pallas as pl
from jax.experimental.pallas import tpu as pltpu
from jax.experimental.pallas import tpu_sc as plsc

VOCAB = 100000
D_MODEL = 1024
NUM_CORES = 2
NUM_SUBCORES = 16
NUM_WORKERS = NUM_CORES * NUM_SUBCORES

CHUNK = 8
NBUF = 8


def _embed_body(n_rows, x_hbm, table_hbm, out_hbm, idx_v, rows_v, gsems, psems):
    b_per_w = n_rows // NUM_WORKERS
    n_chunks = b_per_w // CHUNK
    wid = lax.axis_index("s") * NUM_CORES + lax.axis_index("c")
    base = wid * b_per_w
    pltpu.sync_copy(x_hbm.at[pl.ds(base, b_per_w)], idx_v)

    def gather(ch, b):
        pltpu.async_copy(
            table_hbm.at[idx_v.at[pl.ds(ch * CHUNK, CHUNK)]],
            rows_v.at[b], gsems.at[b],
        )

    def put(ch, b):
        pltpu.async_copy(
            rows_v.at[b], out_hbm.at[pl.ds(base + ch * CHUNK, CHUNK)],
            psems.at[b],
        )

    for b in range(NBUF):
        gather(b, b)

    def wait_gather(ch, b):
        pltpu.make_async_copy(
            table_hbm.at[idx_v.at[pl.ds(ch * CHUNK, CHUNK)]],
            rows_v.at[b], gsems.at[b],
        ).wait()

    def wait_put(ch, b):
        pltpu.make_async_copy(
            rows_v.at[b], out_hbm.at[pl.ds(base + ch * CHUNK, CHUNK)],
            psems.at[b],
        ).wait()

    @pl.loop(0, n_chunks - NBUF, step=NBUF)
    def _chunks(c0):
        for step in range(NBUF + 2):
            if step < NBUF:
                wait_gather(c0 + step, step)
                put(c0 + step, step)
            if step >= 2:
                b = step - 2
                wait_put(c0 + b, b)
                gather(c0 + b + NBUF, b)

    c0 = n_chunks - NBUF
    for b in range(NBUF):
        wait_gather(c0 + b, b)
        put(c0 + b, b)
    for b in range(NBUF):
        wait_put(c0 + b, b)


def kernel(x, table):
    B, T = x.shape
    n_rows = B * T
    x_flat = x.reshape(n_rows).astype(jnp.int32)

    mesh = plsc.VectorSubcoreMesh(
        core_axis_name="c", subcore_axis_name="s",
        num_cores=NUM_CORES, num_subcores=NUM_SUBCORES,
    )
    b_per_w = n_rows // NUM_WORKERS
    run = pl.kernel(
        functools.partial(_embed_body, n_rows),
        out_type=jax.ShapeDtypeStruct((n_rows, D_MODEL), jnp.float32),
        mesh=mesh,
        scratch_types=[
            pltpu.VMEM((b_per_w,), jnp.int32),
            pltpu.VMEM((NBUF, CHUNK, D_MODEL), jnp.float32),
            pltpu.SemaphoreType.DMA((NBUF,)),
            pltpu.SemaphoreType.DMA((NBUF,)),
        ],
    )
    out = run(x_flat, table)
    return out.reshape(B, T, D_MODEL)
